# baseline (speedup 1.0000x reference)
import functools

import jax
import jax.numpy as jnp
from jax.experimental import pallas as pl
from jax.experimental.pallas import tpu as pltpu

_BATCH = 1024
_D = 16
_BLOCK_N = 2048
_NBUF = 4


def _fused_kernel(n_items, user_ref, w_ref, b_ref, gamma_ref, beta_ref,
                  item_ref, out_hbm, h_ref, out_bufs, sems):
    i = pl.program_id(0)
    nsteps = pl.num_programs(0)
    buf = i % _NBUF
    tail = n_items % _BLOCK_N
    tail = _BLOCK_N if tail == 0 else ((tail + 127) // 128) * 128

    @pl.when(i == 0)
    def _():
        h = jnp.dot(user_ref[:], w_ref[:].T,
                    preferred_element_type=jnp.float32) + b_ref[:]
        mu = jnp.mean(h, axis=-1, keepdims=True)
        d = h - mu
        var = jnp.mean(d * d, axis=-1, keepdims=True)
        h_ref[:] = d * jax.lax.rsqrt(var + 1e-5) * gamma_ref[:] + beta_ref[:]

    @pl.when(i >= _NBUF)
    def _():
        pltpu.make_async_copy(
            out_bufs.at[buf],
            out_hbm.at[:, pl.ds((i - _NBUF) * _BLOCK_N, _BLOCK_N)],
            sems.at[buf]).wait()

    out_bufs[buf] = jax.lax.dot_general(
        h_ref[:], item_ref[:], (((1,), (1,)), ((), ())),
        preferred_element_type=jnp.float32)

    @pl.when(i < nsteps - 1)
    def _():
        pltpu.make_async_copy(
            out_bufs.at[buf],
            out_hbm.at[:, pl.ds(i * _BLOCK_N, _BLOCK_N)],
            sems.at[buf]).start()

    @pl.when(i == nsteps - 1)
    def _():
        pltpu.make_async_copy(
            out_bufs.at[buf, :, pl.ds(0, tail)],
            out_hbm.at[:, pl.ds(i * _BLOCK_N, tail)],
            sems.at[buf]).start()
        for k in range(1, _NBUF):
            pltpu.make_async_copy(
                out_bufs.at[(buf + k) % _NBUF],
                out_hbm.at[:, pl.ds((i - _NBUF + k) * _BLOCK_N, _BLOCK_N)],
                sems.at[(buf + k) % _NBUF]).wait()
        pltpu.make_async_copy(
            out_bufs.at[buf, :, pl.ds(0, tail)],
            out_hbm.at[:, pl.ds(i * _BLOCK_N, tail)],
            sems.at[buf]).wait()


@jax.jit
def kernel(user_emb, item_emb, W, b, gamma, beta):
    n_items = item_emb.shape[0]
    grid = (pl.cdiv(n_items, _BLOCK_N),)
    b2 = b.reshape(1, _D)
    gamma2 = gamma.reshape(1, _D)
    beta2 = beta.reshape(1, _D)
    return pl.pallas_call(
        functools.partial(_fused_kernel, n_items),
        grid=grid,
        in_specs=[
            pl.BlockSpec((_BATCH, _D), lambda i: (0, 0)),
            pl.BlockSpec((_D, _D), lambda i: (0, 0)),
            pl.BlockSpec((1, _D), lambda i: (0, 0)),
            pl.BlockSpec((1, _D), lambda i: (0, 0)),
            pl.BlockSpec((1, _D), lambda i: (0, 0)),
            pl.BlockSpec((_BLOCK_N, _D), lambda i: (i, 0)),
        ],
        out_specs=pl.BlockSpec(memory_space=pl.ANY),
        out_shape=jax.ShapeDtypeStruct((_BATCH, n_items), jnp.float32),
        scratch_shapes=[
            pltpu.VMEM((_BATCH, _D), jnp.float32),
            pltpu.VMEM((_NBUF, _BATCH, _BLOCK_N), jnp.float32),
            pltpu.SemaphoreType.DMA((_NBUF,)),
        ],
        compiler_params=pltpu.CompilerParams(
            dimension_semantics=("arbitrary",)),
    )(user_emb, W, b2, gamma2, beta2, item_emb)


# speedup vs baseline: 4.0405x; 4.0405x over previous
import functools

import jax
import jax.numpy as jnp
from jax.experimental import pallas as pl
from jax.experimental.pallas import tpu as pltpu

_BATCH = 1024
_D = 16
_BLOCK_N = 2048


def _fused_kernel(user_ref, w_ref, b_ref, gamma_ref, beta_ref, item_t_ref,
                  out_ref):
    h = jnp.dot(user_ref[:], w_ref[:].T,
                preferred_element_type=jnp.float32) + b_ref[:]
    mu = jnp.mean(h, axis=-1, keepdims=True)
    d = h - mu
    var = jnp.mean(d * d, axis=-1, keepdims=True)
    h = d * jax.lax.rsqrt(var + 1e-5) * gamma_ref[:] + beta_ref[:]
    out_ref[:] = jax.lax.dot_general(
        item_t_ref[:], h, (((0,), (1,)), ((), ())),
        preferred_element_type=jnp.float32)


@jax.jit
def kernel(user_emb, item_emb, W, b, gamma, beta):
    n_items = item_emb.shape[0]
    item_t = item_emb.T
    grid = (pl.cdiv(n_items, _BLOCK_N),)
    b2 = b.reshape(1, _D)
    gamma2 = gamma.reshape(1, _D)
    beta2 = beta.reshape(1, _D)
    out_t = pl.pallas_call(
        _fused_kernel,
        grid=grid,
        in_specs=[
            pl.BlockSpec((_BATCH, _D), lambda i: (0, 0)),
            pl.BlockSpec((_D, _D), lambda i: (0, 0)),
            pl.BlockSpec((1, _D), lambda i: (0, 0)),
            pl.BlockSpec((1, _D), lambda i: (0, 0)),
            pl.BlockSpec((1, _D), lambda i: (0, 0)),
            pl.BlockSpec((_D, _BLOCK_N), lambda i: (0, i)),
        ],
        out_specs=pl.BlockSpec((_BLOCK_N, _BATCH), lambda i: (i, 0)),
        out_shape=jax.ShapeDtypeStruct((n_items, _BATCH), jnp.float32),
        compiler_params=pltpu.CompilerParams(
            dimension_semantics=("parallel",)),
    )(user_emb, W, b2, gamma2, beta2, item_t)
    return out_t.T
